# baseline (speedup 1.0000x reference)
import functools

import jax
import jax.numpy as jnp
from jax import lax
from jax.experimental import pallas as pl
from jax.experimental.pallas import tpu as pltpu
from jax.experimental.pallas import tpu_sc as plsc

_N_DAYS = 365
_N_BUCKETS = 1024
_N_RBF = 16
_N_BASES = 1 + _N_RBF
_N = 16384
_PLANE = _N_DAYS * _N_BUCKETS
_PSEG = _PLANE // 16

_L = 16
_NC = 2
_NS = 16
_NW = _NC * _NS
_B_PER_W = _N // _NW
_CHUNK = 128
_N_CHUNKS = _B_PER_W // _CHUNK
_G_PER_CHUNK = _CHUNK // _L
_WIN = 6

_INV_SIGMA = 20.0

_mesh = plsc.VectorSubcoreMesh(core_axis_name="c", subcore_axis_name="s")


def _make_half(first_half):
    nstream = 1 + _WIN if first_half else _WIN

    @functools.partial(
        pl.kernel,
        mesh=_mesh,
        compiler_params=pltpu.CompilerParams(
            needs_layout_passes=False, use_tc_tiling_on_sc=False
        ),
        out_type=jax.ShapeDtypeStruct((_N,), jnp.float32),
        scratch_types=[
            pltpu.VMEM((_B_PER_W,), jnp.float32),
            pltpu.VMEM((_B_PER_W,), jnp.int32),
            pltpu.VMEM((_B_PER_W,), jnp.int32),
            pltpu.VMEM((_B_PER_W,), jnp.int32),
            pltpu.VMEM((_B_PER_W,), jnp.int32),
            pltpu.VMEM((_B_PER_W,), jnp.float32),
            pltpu.VMEM((_N_CHUNKS * nstream, _CHUNK), jnp.int32),
            [[pltpu.VMEM((_CHUNK, _L), jnp.float32)
              for _ in range(nstream)] for _ in range(_N_CHUNKS)],
            pltpu.VMEM((_B_PER_W,), jnp.float32),
            [pltpu.SemaphoreType.DMA for _ in range(_N_CHUNKS)],
        ],
    )
    def half(mvoc_hbm, day_hbm, bucket_hbm, w_hbm, aux_hbm, out_hbm,
             mvoc_v, day_v, bucket_v, flat_v, win_v, part_v, sidx_v,
             seg_vs, out_v, sems):
        wid = lax.axis_index("s") * _NC + lax.axis_index("c")
        base = wid * _B_PER_W

        pltpu.sync_copy(mvoc_hbm.at[pl.ds(base, _B_PER_W)], mvoc_v)
        pltpu.sync_copy(day_hbm.at[pl.ds(base, _B_PER_W)], day_v)
        pltpu.sync_copy(bucket_hbm.at[pl.ds(base, _B_PER_W)], bucket_v)
        if not first_half:
            pltpu.sync_copy(aux_hbm.at[pl.ds(base, _B_PER_W)], part_v)

        copies = []
        for j in range(_N_CHUNKS):
            for g in range(_G_PER_CHUNK):
                o = j * _CHUNK + g * _L
                d = day_v[pl.ds(o, _L)]
                b = bucket_v[pl.ds(o, _L)]
                flat = d * _N_BUCKETS + b
                flat_v[pl.ds(o, _L)] = flat
                z = mvoc_v[pl.ds(o, _L)]
                z = jnp.minimum(jnp.maximum(z, 0.0), 1.0)
                m = (z * 16.0 - 0.5).astype(jnp.int32)
                s = jnp.minimum(jnp.maximum(m - 2, 0), 16 - _WIN)
                win_v[pl.ds(o, _L)] = s
                seg0 = flat >> 4
                for t in range(nstream):
                    if first_half:
                        if t == 0:
                            seg = seg0
                        else:
                            local = jnp.minimum(s + t, 8) - 1
                            seg = local * _PSEG + seg0
                    else:
                        local = jnp.maximum(s + (t + 1) - 9, 0)
                        seg = local * _PSEG + seg0
                    sidx_v[j * nstream + t, pl.ds(g * _L, _L)] = seg
            copies.append([pltpu.async_copy(
                (aux_hbm if (first_half and t == 0) else w_hbm
                 ).at[sidx_v.at[j * nstream + t]], seg_vs[j][t], sems[j])
                for t in range(nstream)])

        for j in range(_N_CHUNKS):
            for c in copies[j]:
                c.wait()
            bufs = seg_vs[j]
            for g in range(_G_PER_CHUNK):
                o = j * _CHUNK + g * _L
                rows = lax.iota(jnp.int32, _L) + g * _L
                lane = flat_v[pl.ds(o, _L)] & 15
                z = mvoc_v[pl.ds(o, _L)]
                z = jnp.minimum(jnp.maximum(z, 0.0), 1.0)
                s = win_v[pl.ds(o, _L)]
                sf = s.astype(jnp.float32)
                d0 = z * _INV_SIGMA - sf * 1.25
                if first_half:
                    acc = plsc.load_gather(bufs[0], [rows, lane])
                    tset = range(1, nstream)
                else:
                    acc = part_v[pl.ds(o, _L)]
                    tset = range(nstream)
                for ti in tset:
                    t = ti if first_half else ti + 1
                    wk = plsc.load_gather(bufs[ti], [rows, lane])
                    diff = d0 - (0.625 + (t - 1) * 1.25)
                    e = jnp.exp(diff * diff * -0.5) * wk
                    keep = (s + t) <= 8 if first_half else (s + t) >= 9
                    acc = acc + jnp.where(keep, e, 0.0)
                out_v[pl.ds(o, _L)] = acc

        pltpu.sync_copy(out_v, out_hbm.at[pl.ds(base, _B_PER_W)])

    return half


_half_a = _make_half(True)
_half_b = _make_half(False)


def kernel(mvoc, day_idx, bucket_idx, weights):
    w0 = weights[:, :, 0].reshape(_PSEG, _L)
    wa = weights[:, :, 1:9].transpose(2, 0, 1).reshape(8 * _PSEG, _L)
    wb = weights[:, :, 9:17].transpose(2, 0, 1).reshape(8 * _PSEG, _L)
    mvoc1 = mvoc.reshape(_N)
    day1 = day_idx.reshape(_N)
    bucket1 = bucket_idx.reshape(_N)
    partial = _half_a(mvoc1, day1, bucket1, wa, w0)
    out = _half_b(mvoc1, day1, bucket1, wb, partial)
    return out.reshape(_N, 1)


# speedup vs baseline: 1.5506x; 1.5506x over previous
import functools

import jax
import jax.numpy as jnp
from jax import lax
from jax.experimental import pallas as pl
from jax.experimental.pallas import tpu as pltpu
from jax.experimental.pallas import tpu_sc as plsc

_N_DAYS = 365
_N_BUCKETS = 1024
_N_RBF = 16
_N_BASES = 1 + _N_RBF
_N = 16384
_PLANE = _N_DAYS * _N_BUCKETS
_SEG_ROWS = _N_BASES * _PLANE // 16

_L = 16
_NC = 2
_NS = 16
_NW = _NC * _NS
_B_PER_W = _N // _NW
_CHUNK = 128
_N_CHUNKS = _B_PER_W // _CHUNK
_G_PER_CHUNK = _CHUNK // _L
_WIN = 6
_NSTREAM = 1 + _WIN

_INV_SIGMA = 20.0

_mesh = plsc.VectorSubcoreMesh(core_axis_name="c", subcore_axis_name="s")


@functools.partial(
    pl.kernel,
    mesh=_mesh,
    compiler_params=pltpu.CompilerParams(
        needs_layout_passes=False, use_tc_tiling_on_sc=False
    ),
    out_type=jax.ShapeDtypeStruct((_N,), jnp.float32),
    scratch_types=[
        pltpu.VMEM((_B_PER_W,), jnp.float32),
        pltpu.VMEM((_B_PER_W,), jnp.int32),
        pltpu.VMEM((_B_PER_W,), jnp.int32),
        pltpu.VMEM((_B_PER_W,), jnp.int32),
        pltpu.VMEM((_B_PER_W,), jnp.int32),
        pltpu.VMEM((_N_CHUNKS * _NSTREAM, _CHUNK), jnp.int32),
        [[pltpu.VMEM((_CHUNK, _L), jnp.float32)
          for _ in range(_NSTREAM)] for _ in range(_N_CHUNKS)],
        pltpu.VMEM((_B_PER_W,), jnp.float32),
        [pltpu.SemaphoreType.DMA for _ in range(_N_CHUNKS)],
    ],
)
def _gauss_delta_sc(mvoc_hbm, day_hbm, bucket_hbm, w_hbm, out_hbm,
                    mvoc_v, day_v, bucket_v, flat_v, win_v, sidx_v,
                    seg_vs, out_v, sems):
    wid = lax.axis_index("s") * _NC + lax.axis_index("c")
    base = wid * _B_PER_W

    pltpu.sync_copy(mvoc_hbm.at[pl.ds(base, _B_PER_W)], mvoc_v)
    pltpu.sync_copy(day_hbm.at[pl.ds(base, _B_PER_W)], day_v)
    pltpu.sync_copy(bucket_hbm.at[pl.ds(base, _B_PER_W)], bucket_v)

    copies = []
    for j in range(_N_CHUNKS):
        for g in range(_G_PER_CHUNK):
            o = j * _CHUNK + g * _L
            d = day_v[pl.ds(o, _L)]
            b = bucket_v[pl.ds(o, _L)]
            flat = d * _N_BUCKETS + b
            flat_v[pl.ds(o, _L)] = flat
            z = mvoc_v[pl.ds(o, _L)]
            z = jnp.minimum(jnp.maximum(z, 0.0), 1.0)
            m = (z * 16.0 - 0.5).astype(jnp.int32)
            s = jnp.minimum(jnp.maximum(m - 2, 0), 16 - _WIN)
            win_v[pl.ds(o, _L)] = s
            seg0 = flat >> 4
            for t in range(_NSTREAM):
                plane = (s + t) if t > 0 else jnp.zeros((_L,), jnp.int32)
                sidx_v[j * _NSTREAM + t, pl.ds(g * _L, _L)] = (
                    plane * (_PLANE // 16) + seg0)
        copies.append([pltpu.async_copy(
            w_hbm.at[sidx_v.at[j * _NSTREAM + t]], seg_vs[j][t], sems[j])
            for t in range(_NSTREAM)])

    for j in range(_N_CHUNKS):
        for c in copies[j]:
            c.wait()
        bufs = seg_vs[j]
        for g in range(_G_PER_CHUNK):
            o = j * _CHUNK + g * _L
            rows = lax.iota(jnp.int32, _L) + g * _L
            lane = flat_v[pl.ds(o, _L)] & 15
            z = mvoc_v[pl.ds(o, _L)]
            z = jnp.minimum(jnp.maximum(z, 0.0), 1.0)
            sf = win_v[pl.ds(o, _L)].astype(jnp.float32)
            d0 = z * _INV_SIGMA - sf * 1.25
            acc = plsc.load_gather(bufs[0], [rows, lane])
            for t in range(1, _NSTREAM):
                wk = plsc.load_gather(bufs[t], [rows, lane])
                diff = d0 - (0.625 + (t - 1) * 1.25)
                acc = acc + jnp.exp(diff * diff * -0.5) * wk
            out_v[pl.ds(o, _L)] = acc

    pltpu.sync_copy(out_v, out_hbm.at[pl.ds(base, _B_PER_W)])


def kernel(mvoc, day_idx, bucket_idx, weights):
    w_seg = weights.transpose(2, 0, 1).reshape(_SEG_ROWS, _L)
    out = _gauss_delta_sc(
        mvoc.reshape(_N),
        day_idx.reshape(_N),
        bucket_idx.reshape(_N),
        w_seg,
    )
    return out.reshape(_N, 1)
